# Initial kernel scaffold; baseline (speedup 1.0000x reference)
import jax
import jax.numpy as jnp
from jax.experimental import pallas as pl


def kernel(x, edge_index, W1_l, b1_l, W1_r, W2_l, b2_l, W2_r, Wh1, bh1, Wh2, bh2):
    raise NotImplementedError("write your pallas kernel here")


# speedup vs baseline: 14.6339x; 14.6339x over previous
import functools

import jax
import jax.numpy as jnp
from jax import lax
from jax.experimental import pallas as pl
from jax.experimental.pallas import tpu as pltpu
from jax.experimental.pallas import tpu_sc as plsc

N_CORES = 2
N_SUBCORES = 16
LANES = 128
G = 8
FEAT = 16


def _seg_sum_body(edge_split, n_pad, n_rows,
                  src_hbm, dst_hbm, t0_hbm, t1_hbm, out0_hbm, out1_hbm,
                  srcv, dstv, rows, accsh, zbuf, gsem, ssem):
    c = lax.axis_index("c")
    s = lax.axis_index("s")

    full_groups = n_rows // G
    rem_rows = n_rows % G
    total_slots = full_groups + (1 if rem_rows else 0)
    if edge_split:
        wid = s * N_CORES + c
        stride = N_CORES * N_SUBCORES
    else:
        wid = s
        stride = N_SUBCORES
    n_k = -(-total_slots // stride)

    tile_rows = n_pad // N_SUBCORES
    zrows = zbuf.shape[0]

    def zbody(i, carry):
        zbuf[i, :] = jnp.zeros((FEAT,), jnp.float32)
        return carry

    lax.fori_loop(0, zrows, zbody, 0)
    tile_base = s * tile_rows
    for kk in range(tile_rows // zrows):
        pltpu.sync_copy(zbuf, accsh.at[pl.ds(tile_base + kk * zrows, zrows)])
    plsc.subcore_barrier()

    def chunk(table, g, nrows):
        r0 = g * G
        pltpu.sync_copy(src_hbm.at[pl.ds(r0, nrows)], srcv.at[pl.ds(0, nrows)])
        pltpu.sync_copy(dst_hbm.at[pl.ds(r0, nrows)], dstv.at[pl.ds(0, nrows)])
        gcps = [pltpu.async_copy(table.at[srcv.at[j]], rows.at[j], gsem)
                for j in range(nrows)]
        for cp in gcps:
            cp.wait()
        scps = [pltpu.async_copy(rows.at[j], accsh.at[dstv.at[j]], ssem, add=True)
                for j in range(nrows)]
        for cp in scps:
            cp.wait()

    def main_loop(table):
        def kbody(k, carry):
            g = wid + stride * k

            @pl.when(g < full_groups)
            def _():
                chunk(table, g, G)

            if rem_rows:
                @pl.when(g == full_groups)
                def _():
                    chunk(table, g, rem_rows)

            return carry

        lax.fori_loop(0, n_k, kbody, 0)

    @pl.when(c == 0)
    def _():
        main_loop(t0_hbm)

    @pl.when(c == 1)
    def _():
        main_loop(t1_hbm)

    plsc.subcore_barrier()

    @pl.when(c == 0)
    def _():
        pltpu.sync_copy(accsh.at[pl.ds(tile_base, tile_rows)],
                        out0_hbm.at[pl.ds(tile_base, tile_rows)])

    @pl.when(c == 1)
    def _():
        pltpu.sync_copy(accsh.at[pl.ds(tile_base, tile_rows)],
                        out1_hbm.at[pl.ds(tile_base, tile_rows)])


def _make_seg_sum(n_pad, n_rows, edge_split):
    mesh = plsc.VectorSubcoreMesh(core_axis_name="c", subcore_axis_name="s",
                                  num_cores=N_CORES, num_subcores=N_SUBCORES)
    tile_rows = n_pad // N_SUBCORES
    assert tile_rows % 16 == 0
    zrows = next(z for z in range(512, 7, -8) if tile_rows % z == 0)
    out = jax.ShapeDtypeStruct((n_pad, FEAT), jnp.float32)
    return pl.kernel(
        functools.partial(_seg_sum_body, edge_split, n_pad, n_rows),
        out_type=(out, out),
        mesh=mesh,
        scratch_types=[
            pltpu.VMEM((G, LANES), jnp.int32),
            pltpu.VMEM((G, LANES), jnp.int32),
            pltpu.VMEM((G, LANES, FEAT), jnp.float32),
            pltpu.VMEM_SHARED((n_pad, FEAT), jnp.float32),
            pltpu.VMEM((zrows, FEAT), jnp.float32),
            pltpu.SemaphoreType.DMA,
            pltpu.SemaphoreType.DMA,
        ],
        compiler_params=pltpu.CompilerParams(use_tc_tiling_on_sc=False),
    )


def _tc_a_body(x_ref, a0_ref, a1_ref, w1l_ref, b1_ref, w1r_ref,
               w2l_ref, b2_ref, w2r_ref,
               plo_ref, phi_ref, r_ref, invd_ref):
    a = a0_ref[...] + a1_ref[...]
    deg = a[:, 8:9]
    invd = 1.0 / jnp.maximum(deg, 1.0)
    agg = a[:, :8] * invd
    xb = x_ref[...]
    h = jnp.dot(agg, w1l_ref[...], preferred_element_type=jnp.float32)
    h += jnp.dot(xb, w1r_ref[...], preferred_element_type=jnp.float32)
    h = jnp.maximum(h + b1_ref[...], 0.0)
    p = jnp.dot(h, w2l_ref[...], preferred_element_type=jnp.float32)
    r = jnp.dot(h, w2r_ref[...], preferred_element_type=jnp.float32) + b2_ref[...]
    plo_ref[...] = p[:, :FEAT]
    phi_ref[...] = p[:, FEAT:]
    r_ref[...] = r
    invd_ref[...] = invd


def _tc_b_body(g0_ref, g1_ref, r_ref, invd_ref, wh1_ref, bh1_ref,
               wh2_ref, bh2_ref, out_ref):
    agg = jnp.concatenate([g0_ref[...], g1_ref[...]], axis=1) * invd_ref[...]
    z = jnp.maximum(agg + r_ref[...], 0.0)
    h1 = jnp.maximum(
        jnp.dot(z, wh1_ref[...], preferred_element_type=jnp.float32)
        + bh1_ref[...], 0.0)
    logits = jnp.dot(h1, wh2_ref[...], preferred_element_type=jnp.float32)
    out_ref[...] = jax.nn.sigmoid(logits + bh2_ref[...])


def kernel(x, edge_index, W1_l, b1_l, W1_r, W2_l, b2_l, W2_r, Wh1, bh1, Wh2, bh2):
    n = x.shape[0]
    e = edge_index.shape[1]
    assert e % LANES == 0
    n_rows = e // LANES
    n_pad = -(-n // 128) * 128

    ei = edge_index.astype(jnp.int32)
    src2d = ei[0].reshape(n_rows, LANES)
    dst2d = ei[1].reshape(n_rows, LANES)
    x_aug = jnp.concatenate(
        [x, jnp.ones((n, 1), jnp.float32), jnp.zeros((n, FEAT - 9), jnp.float32)],
        axis=1)

    seg1 = _make_seg_sum(n_pad, n_rows, edge_split=True)
    a0, a1 = seg1(src2d, dst2d, x_aug, x_aug)
    a0, a1 = a0[:n], a1[:n]

    bn = 2000
    grid = (n // bn,)
    row_spec = lambda w: pl.BlockSpec((bn, w), lambda i: (i, 0))
    full_spec = lambda shape: pl.BlockSpec(shape, lambda i: (0, 0))

    p_lo, p_hi, r, invd = pl.pallas_call(
        _tc_a_body,
        grid=grid,
        in_specs=[
            row_spec(8), row_spec(FEAT), row_spec(FEAT),
            full_spec((8, 64)), full_spec((1, 64)), full_spec((8, 64)),
            full_spec((64, 32)), full_spec((1, 32)), full_spec((64, 32)),
        ],
        out_specs=[row_spec(FEAT), row_spec(FEAT), row_spec(32), row_spec(1)],
        out_shape=[
            jax.ShapeDtypeStruct((n, FEAT), jnp.float32),
            jax.ShapeDtypeStruct((n, FEAT), jnp.float32),
            jax.ShapeDtypeStruct((n, 32), jnp.float32),
            jax.ShapeDtypeStruct((n, 1), jnp.float32),
        ],
    )(x, a0, a1, W1_l.T, b1_l.reshape(1, 64), W1_r.T,
      W2_l.T, b2_l.reshape(1, 32), W2_r.T)

    seg2 = _make_seg_sum(n_pad, n_rows, edge_split=False)
    g0, g1 = seg2(src2d, dst2d, p_lo, p_hi)
    g0, g1 = g0[:n], g1[:n]

    probs = pl.pallas_call(
        _tc_b_body,
        grid=grid,
        in_specs=[
            row_spec(FEAT), row_spec(FEAT), row_spec(32), row_spec(1),
            full_spec((32, 16)), full_spec((1, 16)),
            full_spec((16, 1)), full_spec((1, 1)),
        ],
        out_specs=[row_spec(1)],
        out_shape=[jax.ShapeDtypeStruct((n, 1), jnp.float32)],
    )(g0, g1, r, invd, Wh1.T, bh1.reshape(1, 16), Wh2.T, bh2.reshape(1, 1))[0]

    return probs[:, 0]


# speedup vs baseline: 16.2402x; 1.1098x over previous
import functools

import jax
import jax.numpy as jnp
from jax import lax
from jax.experimental import pallas as pl
from jax.experimental.pallas import tpu as pltpu
from jax.experimental.pallas import tpu_sc as plsc

N_CORES = 2
N_SUBCORES = 16
LANES = 128
G = 8
FEAT = 16


def _seg_sum_body(edge_split, n_pad, n_rows,
                  src_hbm, dst_hbm, t0_hbm, t1_hbm, out0_hbm, out1_hbm,
                  srcv, dstv, rows, accsh, zbuf, gsem, ssem):
    c = lax.axis_index("c")
    s = lax.axis_index("s")

    full_groups = n_rows // G
    rem_rows = n_rows % G
    total_slots = full_groups + (1 if rem_rows else 0)
    if edge_split:
        wid = s * N_CORES + c
        stride = N_CORES * N_SUBCORES
    else:
        wid = s
        stride = N_SUBCORES
    n_k = -(-total_slots // stride)

    tile_rows = n_pad // N_SUBCORES
    zrows = zbuf.shape[0]

    def zbody(i, carry):
        zbuf[i, :] = jnp.zeros((FEAT,), jnp.float32)
        return carry

    lax.fori_loop(0, zrows, zbody, 0)
    tile_base = s * tile_rows
    for kk in range(tile_rows // zrows):
        pltpu.sync_copy(zbuf, accsh.at[pl.ds(tile_base + kk * zrows, zrows)])
    plsc.subcore_barrier()

    def chunk(table, g, nrows):
        r0 = g * G
        pltpu.sync_copy(src_hbm.at[pl.ds(r0, nrows)], srcv.at[pl.ds(0, nrows)])
        pltpu.sync_copy(dst_hbm.at[pl.ds(r0, nrows)], dstv.at[pl.ds(0, nrows)])
        gcps = [pltpu.async_copy(table.at[srcv.at[j]], rows.at[j], gsem)
                for j in range(nrows)]
        for cp in gcps:
            cp.wait()
        scps = [pltpu.async_copy(rows.at[j], accsh.at[dstv.at[j]], ssem, add=True)
                for j in range(nrows)]
        for cp in scps:
            cp.wait()

    def main_loop(table):
        def kbody(k, carry):
            g = wid + stride * k

            @pl.when(g < full_groups)
            def _():
                chunk(table, g, G)

            if rem_rows:
                @pl.when(g == full_groups)
                def _():
                    chunk(table, g, rem_rows)

            return carry

        lax.fori_loop(0, n_k, kbody, 0)

    @pl.when(c == 0)
    def _():
        main_loop(t0_hbm)

    @pl.when(c == 1)
    def _():
        main_loop(t1_hbm)

    plsc.subcore_barrier()

    @pl.when(c == 0)
    def _():
        pltpu.sync_copy(accsh.at[pl.ds(tile_base, tile_rows)],
                        out0_hbm.at[pl.ds(tile_base, tile_rows)])

    @pl.when(c == 1)
    def _():
        pltpu.sync_copy(accsh.at[pl.ds(tile_base, tile_rows)],
                        out1_hbm.at[pl.ds(tile_base, tile_rows)])


def _make_seg_sum(n_pad, n_rows, edge_split):
    mesh = plsc.VectorSubcoreMesh(core_axis_name="c", subcore_axis_name="s",
                                  num_cores=N_CORES, num_subcores=N_SUBCORES)
    tile_rows = n_pad // N_SUBCORES
    assert tile_rows % 16 == 0
    zrows = next(z for z in range(512, 7, -8) if tile_rows % z == 0)
    out = jax.ShapeDtypeStruct((n_pad, FEAT), jnp.float32)
    return pl.kernel(
        functools.partial(_seg_sum_body, edge_split, n_pad, n_rows),
        out_type=(out, out),
        mesh=mesh,
        scratch_types=[
            pltpu.VMEM((G, LANES), jnp.int32),
            pltpu.VMEM((G, LANES), jnp.int32),
            pltpu.VMEM((G, LANES, FEAT), jnp.float32),
            pltpu.VMEM_SHARED((n_pad, FEAT), jnp.float32),
            pltpu.VMEM((zrows, FEAT), jnp.float32),
            pltpu.SemaphoreType.DMA,
            pltpu.SemaphoreType.DMA,
        ],
        compiler_params=pltpu.CompilerParams(use_tc_tiling_on_sc=False),
    )


def _bdot(a, b):
    return jnp.dot(a.astype(jnp.bfloat16), b, preferred_element_type=jnp.float32)


def _tc_a_body(x_ref, a0_ref, a1_ref, w1l_ref, b1_ref, w1r_ref,
               w2l_ref, b2_ref, w2r_ref,
               plo_ref, phi_ref, r_ref, invd_ref):
    a = a0_ref[...] + a1_ref[...]
    deg = a[:, 8:9]
    invd = 1.0 / jnp.maximum(deg, 1.0)
    agg = a[:, :8] * invd
    xb = x_ref[:, :8]
    h = _bdot(agg, w1l_ref[...]) + _bdot(xb, w1r_ref[...])
    h = jnp.maximum(h + b1_ref[...], 0.0)
    p = _bdot(h, w2l_ref[...])
    r = _bdot(h, w2r_ref[...]) + b2_ref[...]
    plo_ref[...] = p[:, :FEAT]
    phi_ref[...] = p[:, FEAT:]
    r_ref[...] = r
    invd_ref[...] = invd


def _tc_b_body(g0_ref, g1_ref, r_ref, invd_ref, wh1_ref, bh1_ref,
               wh2_ref, bh2_ref, out_ref):
    agg = jnp.concatenate([g0_ref[...], g1_ref[...]], axis=1) * invd_ref[...]
    z = jnp.maximum(agg + r_ref[...], 0.0)
    h1 = jnp.maximum(_bdot(z, wh1_ref[...]) + bh1_ref[...], 0.0)
    logits = _bdot(h1, wh2_ref[...])
    out_ref[...] = jax.nn.sigmoid(logits + bh2_ref[...])


def kernel(x, edge_index, W1_l, b1_l, W1_r, W2_l, b2_l, W2_r, Wh1, bh1, Wh2, bh2):
    n = x.shape[0]
    e = edge_index.shape[1]
    assert e % LANES == 0
    n_rows = e // LANES
    n_pad = -(-n // 128) * 128

    ei = edge_index.astype(jnp.int32)
    src2d = ei[0].reshape(n_rows, LANES)
    dst2d = ei[1].reshape(n_rows, LANES)
    x_aug = jnp.concatenate(
        [x, jnp.ones((n, 1), jnp.float32), jnp.zeros((n, FEAT - 9), jnp.float32)],
        axis=1)
    x_aug = jnp.concatenate([x_aug, jnp.zeros((n_pad - n, FEAT), jnp.float32)],
                            axis=0)

    seg1 = _make_seg_sum(n_pad, n_rows, edge_split=True)
    a0, a1 = seg1(src2d, dst2d, x_aug, x_aug)

    bn = 3128
    assert n_pad % bn == 0
    grid = (n_pad // bn,)
    row_spec = lambda w: pl.BlockSpec((bn, w), lambda i: (i, 0))
    full_spec = lambda shape: pl.BlockSpec(shape, lambda i: (0, 0))
    bf = lambda w: w.astype(jnp.bfloat16)

    p_lo, p_hi, r, invd = pl.pallas_call(
        _tc_a_body,
        grid=grid,
        in_specs=[
            row_spec(FEAT), row_spec(FEAT), row_spec(FEAT),
            full_spec((8, 64)), full_spec((1, 64)), full_spec((8, 64)),
            full_spec((64, 32)), full_spec((1, 32)), full_spec((64, 32)),
        ],
        out_specs=[row_spec(FEAT), row_spec(FEAT), row_spec(32), row_spec(1)],
        out_shape=[
            jax.ShapeDtypeStruct((n_pad, FEAT), jnp.float32),
            jax.ShapeDtypeStruct((n_pad, FEAT), jnp.float32),
            jax.ShapeDtypeStruct((n_pad, 32), jnp.float32),
            jax.ShapeDtypeStruct((n_pad, 1), jnp.float32),
        ],
    )(x_aug, a0, a1, bf(W1_l.T), b1_l.reshape(1, 64), bf(W1_r.T),
      bf(W2_l.T), b2_l.reshape(1, 32), bf(W2_r.T))

    seg2 = _make_seg_sum(n_pad, n_rows, edge_split=False)
    g0, g1 = seg2(src2d, dst2d, p_lo, p_hi)

    probs = pl.pallas_call(
        _tc_b_body,
        grid=grid,
        in_specs=[
            row_spec(FEAT), row_spec(FEAT), row_spec(32), row_spec(1),
            full_spec((32, 16)), full_spec((1, 16)),
            full_spec((16, 1)), full_spec((1, 1)),
        ],
        out_specs=[row_spec(1)],
        out_shape=[jax.ShapeDtypeStruct((n_pad, 1), jnp.float32)],
    )(g0, g1, r, invd, bf(Wh1.T), bh1.reshape(1, 16), bf(Wh2.T),
      bh2.reshape(1, 1))[0]

    return probs[:n, 0]


# speedup vs baseline: 20.4672x; 1.2603x over previous
import functools

import jax
import jax.numpy as jnp
from jax import lax
from jax.experimental import pallas as pl
from jax.experimental.pallas import tpu as pltpu
from jax.experimental.pallas import tpu_sc as plsc

N_CORES = 2
N_SUBCORES = 16
LANES = 128
G = 8
FEAT = 16


def _seg_sum_body(edge_split, n_pad, n_rows,
                  src_hbm, dst_hbm, t0_hbm, t1_hbm, out0_hbm, out1_hbm,
                  srcv, dstv, rows, accsh, zbuf, gsem, ssem):
    c = lax.axis_index("c")
    s = lax.axis_index("s")

    full_groups = n_rows // G
    rem_rows = n_rows % G
    total_slots = full_groups + (1 if rem_rows else 0)
    if edge_split:
        wid = s * N_CORES + c
        stride = N_CORES * N_SUBCORES
    else:
        wid = s
        stride = N_SUBCORES
    n_k = -(-total_slots // stride)

    tile_rows = n_pad // N_SUBCORES
    zrows = zbuf.shape[0]

    def zbody(i, carry):
        zbuf[i, :] = jnp.zeros((FEAT,), jnp.float32)
        return carry

    lax.fori_loop(0, zrows, zbody, 0)
    tile_base = s * tile_rows
    for kk in range(tile_rows // zrows):
        pltpu.sync_copy(zbuf, accsh.at[pl.ds(tile_base + kk * zrows, zrows)])
    plsc.subcore_barrier()

    def chunk(table, g, nrows):
        r0 = g * G
        pltpu.sync_copy(src_hbm.at[pl.ds(r0, nrows)], srcv.at[pl.ds(0, nrows)])
        pltpu.sync_copy(dst_hbm.at[pl.ds(r0, nrows)], dstv.at[pl.ds(0, nrows)])
        gcps = [pltpu.async_copy(table.at[srcv.at[j]], rows.at[j], gsem)
                for j in range(nrows)]
        for cp in gcps:
            cp.wait()
        scps = [pltpu.async_copy(rows.at[j], accsh.at[dstv.at[j]], ssem, add=True)
                for j in range(nrows)]
        for cp in scps:
            cp.wait()

    def main_loop(table):
        def kbody(k, carry):
            g = wid + stride * k

            @pl.when(g < full_groups)
            def _():
                chunk(table, g, G)

            if rem_rows:
                @pl.when(g == full_groups)
                def _():
                    chunk(table, g, rem_rows)

            return carry

        lax.fori_loop(0, n_k, kbody, 0)

    @pl.when(c == 0)
    def _():
        main_loop(t0_hbm)

    @pl.when(c == 1)
    def _():
        main_loop(t1_hbm)

    plsc.subcore_barrier()

    @pl.when(c == 0)
    def _():
        pltpu.sync_copy(accsh.at[pl.ds(tile_base, tile_rows)],
                        out0_hbm.at[pl.ds(tile_base, tile_rows)])

    @pl.when(c == 1)
    def _():
        pltpu.sync_copy(accsh.at[pl.ds(tile_base, tile_rows)],
                        out1_hbm.at[pl.ds(tile_base, tile_rows)])


def _make_seg_sum(n_pad, n_rows, edge_split):
    mesh = plsc.VectorSubcoreMesh(core_axis_name="c", subcore_axis_name="s",
                                  num_cores=N_CORES, num_subcores=N_SUBCORES)
    tile_rows = n_pad // N_SUBCORES
    assert tile_rows % 16 == 0
    zrows = next(z for z in range(512, 7, -8) if tile_rows % z == 0)
    out = jax.ShapeDtypeStruct((n_pad, FEAT), jnp.float32)
    return pl.kernel(
        functools.partial(_seg_sum_body, edge_split, n_pad, n_rows),
        out_type=(out, out),
        mesh=mesh,
        scratch_types=[
            pltpu.VMEM((G, LANES), jnp.int32),
            pltpu.VMEM((G, LANES), jnp.int32),
            pltpu.VMEM((G, LANES, FEAT), jnp.float32),
            pltpu.VMEM_SHARED((n_pad, FEAT), jnp.float32),
            pltpu.VMEM((zrows, FEAT), jnp.float32),
            pltpu.SemaphoreType.DMA,
            pltpu.SemaphoreType.DMA,
        ],
        compiler_params=pltpu.CompilerParams(use_tc_tiling_on_sc=False),
    )


def _bdot(a, b):
    return jnp.dot(a.astype(jnp.bfloat16), b, preferred_element_type=jnp.float32)


def _pack_body(xr_ref, out_ref):
    bR = out_ref.shape[0]
    out_ref[...] = jnp.zeros((bR, 128), jnp.float32)
    ones = jnp.ones((bR, 1), jnp.float32)
    for g in range(8):
        out_ref[:, 16 * g:16 * g + 8] = xr_ref[:, 8 * g:8 * g + 8]
        out_ref[:, 16 * g + 8:16 * g + 9] = ones


def _tc_a_body(xa_ref, a0_ref, a1_ref, w1l_ref, b1_ref, w1r_ref,
               w2l_ref, b2_ref, w2r_ref,
               plo_ref, phi_ref, r_ref):
    a = a0_ref[...] + a1_ref[...]
    for g in range(8):
        sub = a[:, 16 * g:16 * g + 16]
        invd = 1.0 / jnp.maximum(sub[:, 8:9], 1.0)
        agg = sub[:, :8] * invd
        xb = xa_ref[:, 16 * g:16 * g + 8]
        h = _bdot(agg, w1l_ref[...]) + _bdot(xb, w1r_ref[...])
        h = jnp.maximum(h + b1_ref[...], 0.0)
        p = _bdot(h, w2l_ref[...])
        plo_ref[:, 16 * g:16 * g + 16] = p[:, :FEAT]
        phi_ref[:, 16 * g:16 * g + 16] = p[:, FEAT:]
        r_ref[:, 32 * g:32 * g + 32] = _bdot(h, w2r_ref[...]) + b2_ref[...]


def _tc_b_body(g0_ref, g1_ref, r_ref, a0_ref, a1_ref, wh1_ref, bh1_ref,
               wh2_ref, bh2_ref, out_ref):
    a = a0_ref[...] + a1_ref[...]
    for g in range(8):
        invd = 1.0 / jnp.maximum(a[:, 16 * g + 8:16 * g + 9], 1.0)
        agg = jnp.concatenate(
            [g0_ref[:, 16 * g:16 * g + 16], g1_ref[:, 16 * g:16 * g + 16]],
            axis=1) * invd
        z = jnp.maximum(agg + r_ref[:, 32 * g:32 * g + 32], 0.0)
        h1 = jnp.maximum(_bdot(z, wh1_ref[...]) + bh1_ref[...], 0.0)
        logits = _bdot(h1, wh2_ref[...]) + bh2_ref[...]
        out_ref[:, g:g + 1] = jax.nn.sigmoid(logits)


def kernel(x, edge_index, W1_l, b1_l, W1_r, W2_l, b2_l, W2_r, Wh1, bh1, Wh2, bh2):
    n = x.shape[0]
    e = edge_index.shape[1]
    assert e % LANES == 0
    n_rows = e // LANES
    n_pad = -(-n // 128) * 128

    ei = edge_index.astype(jnp.int32)
    src2d = ei[0].reshape(n_rows, LANES)
    dst2d = ei[1].reshape(n_rows, LANES)

    rp = n_pad // 8
    br = 3128
    assert rp % br == 0
    grid = (rp // br,)
    row_spec = lambda w: pl.BlockSpec((br, w), lambda i: (i, 0))
    full_spec = lambda shape: pl.BlockSpec(shape, lambda i: (0, 0))
    bf = lambda w: w.astype(jnp.bfloat16)

    xr = x.reshape(n // 8, 64)
    xr = jnp.concatenate(
        [xr, jnp.zeros((rp - n // 8, 64), jnp.float32)], axis=0)
    xa_p = pl.pallas_call(
        _pack_body,
        grid=grid,
        in_specs=[row_spec(64)],
        out_specs=row_spec(128),
        out_shape=jax.ShapeDtypeStruct((rp, 128), jnp.float32),
    )(xr)
    xa = xa_p.reshape(n_pad, FEAT)

    seg1 = _make_seg_sum(n_pad, n_rows, edge_split=True)
    a0, a1 = seg1(src2d, dst2d, xa, xa)
    a0_p = a0.reshape(rp, 128)
    a1_p = a1.reshape(rp, 128)

    p_lo, p_hi, r = pl.pallas_call(
        _tc_a_body,
        grid=grid,
        in_specs=[
            row_spec(128), row_spec(128), row_spec(128),
            full_spec((8, 64)), full_spec((1, 64)), full_spec((8, 64)),
            full_spec((64, 32)), full_spec((1, 32)), full_spec((64, 32)),
        ],
        out_specs=[row_spec(128), row_spec(128), row_spec(256)],
        out_shape=[
            jax.ShapeDtypeStruct((rp, 128), jnp.float32),
            jax.ShapeDtypeStruct((rp, 128), jnp.float32),
            jax.ShapeDtypeStruct((rp, 256), jnp.float32),
        ],
    )(xa_p, a0_p, a1_p, bf(W1_l.T), b1_l.reshape(1, 64), bf(W1_r.T),
      bf(W2_l.T), b2_l.reshape(1, 32), bf(W2_r.T))

    seg2 = _make_seg_sum(n_pad, n_rows, edge_split=False)
    g0, g1 = seg2(src2d, dst2d,
                  p_lo.reshape(n_pad, FEAT), p_hi.reshape(n_pad, FEAT))

    out8 = pl.pallas_call(
        _tc_b_body,
        grid=grid,
        in_specs=[
            row_spec(128), row_spec(128), row_spec(256),
            row_spec(128), row_spec(128),
            full_spec((32, 16)), full_spec((1, 16)),
            full_spec((16, 1)), full_spec((1, 1)),
        ],
        out_specs=pl.BlockSpec((br, 8), lambda i: (i, 0)),
        out_shape=jax.ShapeDtypeStruct((rp, 8), jnp.float32),
    )(g0.reshape(rp, 128), g1.reshape(rp, 128), r, a0_p, a1_p,
      bf(Wh1.T), bh1.reshape(1, 16), bf(Wh2.T), bh2.reshape(1, 1))

    return out8.reshape(n_pad)[:n]


# speedup vs baseline: 26.2164x; 1.2809x over previous
import functools

import jax
import jax.numpy as jnp
from jax import lax
from jax.experimental import pallas as pl
from jax.experimental.pallas import tpu as pltpu
from jax.experimental.pallas import tpu_sc as plsc

N_CORES = 2
N_SUBCORES = 16
LANES = 128
G = 5
FEAT = 16


def _seg_sum_body(edge_split, n_pad, n_rows,
                  src_hbm, dst_hbm, t0_hbm, t1_hbm, out0_hbm, out1_hbm,
                  srcv0, dstv0, rows0, srcv1, dstv1, rows1,
                  accsh, zbuf,
                  isem0, gsem0, ssem0, isem1, gsem1, ssem1):
    c = lax.axis_index("c")
    s = lax.axis_index("s")

    assert n_rows % G == 0
    total_slots = n_rows // G
    if edge_split:
        wid = s * N_CORES + c
        stride = N_CORES * N_SUBCORES
    else:
        wid = s
        stride = N_SUBCORES
    n_k = -(-total_slots // stride)
    n_k2 = -(-n_k // 2)

    srcv = (srcv0, srcv1)
    dstv = (dstv0, dstv1)
    rows = (rows0, rows1)
    isem = (isem0, isem1)
    gsem = (gsem0, gsem1)
    ssem = (ssem0, ssem1)

    tile_rows = n_pad // N_SUBCORES
    zrows = zbuf.shape[0]

    def zbody(i, carry):
        zbuf[i, :] = jnp.zeros((FEAT,), jnp.float32)
        return carry

    lax.fori_loop(0, zrows, zbody, 0)
    tile_base = s * tile_rows
    for kk in range(tile_rows // zrows):
        pltpu.sync_copy(zbuf, accsh.at[pl.ds(tile_base + kk * zrows, zrows)])
    plsc.subcore_barrier()

    def idx_issue(g, p):
        r0 = g * G
        pltpu.async_copy(src_hbm.at[pl.ds(r0, G)], srcv[p], isem[p])
        pltpu.async_copy(dst_hbm.at[pl.ds(r0, G)], dstv[p], isem[p])

    def idx_drain(p):
        pltpu.make_async_copy(src_hbm.at[pl.ds(0, G)], srcv[p], isem[p]).wait()
        pltpu.make_async_copy(dst_hbm.at[pl.ds(0, G)], dstv[p], isem[p]).wait()

    def gather_issue(table, p):
        for j in range(G):
            pltpu.async_copy(table.at[srcv[p].at[j]], rows[p].at[j], gsem[p])

    def gather_drain(table, p):
        for j in range(G):
            pltpu.make_async_copy(table.at[srcv[p].at[j]], rows[p].at[j],
                                  gsem[p]).wait()

    def scatter_issue(p):
        for j in range(G):
            pltpu.async_copy(rows[p].at[j], accsh.at[dstv[p].at[j]], ssem[p],
                             add=True)

    def scatter_drain(p):
        for j in range(G):
            pltpu.make_async_copy(rows[p].at[j], accsh.at[dstv[p].at[j]],
                                  ssem[p]).wait()

    def main_loop(table):
        @pl.when(wid < total_slots)
        def _():
            idx_issue(wid, 0)

        def kbody(k2, carry):
            for p in (0, 1):
                k = 2 * k2 + p
                g = wid + stride * k
                q = 1 - p
                g_prev = g - stride
                g_next = g + stride

                @pl.when(g < total_slots)
                def _():
                    idx_drain(p)
                    gather_issue(table, p)

                @pl.when((k >= 1) & (g_prev < total_slots))
                def _():
                    scatter_drain(q)

                @pl.when(g_next < total_slots)
                def _():
                    idx_issue(g_next, q)

                @pl.when(g < total_slots)
                def _():
                    gather_drain(table, p)
                    scatter_issue(p)

            return carry

        lax.fori_loop(0, n_k2, kbody, 0)

        k_last = 2 * n_k2 - 1

        @pl.when(wid + stride * k_last < total_slots)
        def _():
            scatter_drain(k_last % 2)

    @pl.when(c == 0)
    def _():
        main_loop(t0_hbm)

    @pl.when(c == 1)
    def _():
        main_loop(t1_hbm)

    plsc.subcore_barrier()

    @pl.when(c == 0)
    def _():
        pltpu.sync_copy(accsh.at[pl.ds(tile_base, tile_rows)],
                        out0_hbm.at[pl.ds(tile_base, tile_rows)])

    @pl.when(c == 1)
    def _():
        pltpu.sync_copy(accsh.at[pl.ds(tile_base, tile_rows)],
                        out1_hbm.at[pl.ds(tile_base, tile_rows)])


def _make_seg_sum(n_pad, n_rows, edge_split):
    mesh = plsc.VectorSubcoreMesh(core_axis_name="c", subcore_axis_name="s",
                                  num_cores=N_CORES, num_subcores=N_SUBCORES)
    tile_rows = n_pad // N_SUBCORES
    assert tile_rows % 16 == 0
    zrows = next(z for z in range(256, 7, -8) if tile_rows % z == 0)
    out = jax.ShapeDtypeStruct((n_pad, FEAT), jnp.float32)
    buf = [
        pltpu.VMEM((G, LANES), jnp.int32),
        pltpu.VMEM((G, LANES), jnp.int32),
        pltpu.VMEM((G, LANES, FEAT), jnp.float32),
    ]
    return pl.kernel(
        functools.partial(_seg_sum_body, edge_split, n_pad, n_rows),
        out_type=(out, out),
        mesh=mesh,
        scratch_types=buf + buf + [
            pltpu.VMEM_SHARED((n_pad, FEAT), jnp.float32),
            pltpu.VMEM((zrows, FEAT), jnp.float32),
        ] + [pltpu.SemaphoreType.DMA] * 6,
        compiler_params=pltpu.CompilerParams(use_tc_tiling_on_sc=False),
    )


def _bdot(a, b):
    return jnp.dot(a.astype(jnp.bfloat16), b, preferred_element_type=jnp.float32)


def _pack_body(xr_ref, out_ref):
    bR = out_ref.shape[0]
    out_ref[...] = jnp.zeros((bR, 128), jnp.float32)
    ones = jnp.ones((bR, 1), jnp.float32)
    for g in range(8):
        out_ref[:, 16 * g:16 * g + 8] = xr_ref[:, 8 * g:8 * g + 8]
        out_ref[:, 16 * g + 8:16 * g + 9] = ones


def _tc_a_body(xa_ref, a0_ref, a1_ref, w1l_ref, b1_ref, w1r_ref,
               w2l_ref, b2_ref, w2r_ref,
               plo_ref, phi_ref, r_ref):
    a = a0_ref[...] + a1_ref[...]
    for g in range(8):
        sub = a[:, 16 * g:16 * g + 16]
        invd = 1.0 / jnp.maximum(sub[:, 8:9], 1.0)
        agg = sub[:, :8] * invd
        xb = xa_ref[:, 16 * g:16 * g + 8]
        h = _bdot(agg, w1l_ref[...]) + _bdot(xb, w1r_ref[...])
        h = jnp.maximum(h + b1_ref[...], 0.0)
        p = _bdot(h, w2l_ref[...])
        plo_ref[:, 16 * g:16 * g + 16] = p[:, :FEAT]
        phi_ref[:, 16 * g:16 * g + 16] = p[:, FEAT:]
        r_ref[:, 32 * g:32 * g + 32] = _bdot(h, w2r_ref[...]) + b2_ref[...]


def _tc_b_body(g0_ref, g1_ref, r_ref, a0_ref, a1_ref, wh1_ref, bh1_ref,
               wh2_ref, bh2_ref, out_ref):
    a = a0_ref[...] + a1_ref[...]
    for g in range(8):
        invd = 1.0 / jnp.maximum(a[:, 16 * g + 8:16 * g + 9], 1.0)
        agg = jnp.concatenate(
            [g0_ref[:, 16 * g:16 * g + 16], g1_ref[:, 16 * g:16 * g + 16]],
            axis=1) * invd
        z = jnp.maximum(agg + r_ref[:, 32 * g:32 * g + 32], 0.0)
        h1 = jnp.maximum(_bdot(z, wh1_ref[...]) + bh1_ref[...], 0.0)
        logits = _bdot(h1, wh2_ref[...]) + bh2_ref[...]
        out_ref[:, g:g + 1] = jax.nn.sigmoid(logits)


def kernel(x, edge_index, W1_l, b1_l, W1_r, W2_l, b2_l, W2_r, Wh1, bh1, Wh2, bh2):
    n = x.shape[0]
    e = edge_index.shape[1]
    assert e % LANES == 0
    n_rows = e // LANES
    n_pad = -(-n // 128) * 128

    ei = edge_index.astype(jnp.int32)
    src2d = ei[0].reshape(n_rows, LANES)
    dst2d = ei[1].reshape(n_rows, LANES)

    rp = n_pad // 8
    br = 3128
    assert rp % br == 0
    grid = (rp // br,)
    row_spec = lambda w: pl.BlockSpec((br, w), lambda i: (i, 0))
    full_spec = lambda shape: pl.BlockSpec(shape, lambda i: (0, 0))
    bf = lambda w: w.astype(jnp.bfloat16)

    xr = x.reshape(n // 8, 64)
    xr = jnp.concatenate(
        [xr, jnp.zeros((rp - n // 8, 64), jnp.float32)], axis=0)
    xa_p = pl.pallas_call(
        _pack_body,
        grid=grid,
        in_specs=[row_spec(64)],
        out_specs=row_spec(128),
        out_shape=jax.ShapeDtypeStruct((rp, 128), jnp.float32),
    )(xr)
    xa = xa_p.reshape(n_pad, FEAT)

    seg1 = _make_seg_sum(n_pad, n_rows, edge_split=True)
    a0, a1 = seg1(src2d, dst2d, xa, xa)
    a0_p = a0.reshape(rp, 128)
    a1_p = a1.reshape(rp, 128)

    p_lo, p_hi, r = pl.pallas_call(
        _tc_a_body,
        grid=grid,
        in_specs=[
            row_spec(128), row_spec(128), row_spec(128),
            full_spec((8, 64)), full_spec((1, 64)), full_spec((8, 64)),
            full_spec((64, 32)), full_spec((1, 32)), full_spec((64, 32)),
        ],
        out_specs=[row_spec(128), row_spec(128), row_spec(256)],
        out_shape=[
            jax.ShapeDtypeStruct((rp, 128), jnp.float32),
            jax.ShapeDtypeStruct((rp, 128), jnp.float32),
            jax.ShapeDtypeStruct((rp, 256), jnp.float32),
        ],
    )(xa_p, a0_p, a1_p, bf(W1_l.T), b1_l.reshape(1, 64), bf(W1_r.T),
      bf(W2_l.T), b2_l.reshape(1, 32), bf(W2_r.T))

    seg2 = _make_seg_sum(n_pad, n_rows, edge_split=False)
    g0, g1 = seg2(src2d, dst2d,
                  p_lo.reshape(n_pad, FEAT), p_hi.reshape(n_pad, FEAT))

    out8 = pl.pallas_call(
        _tc_b_body,
        grid=grid,
        in_specs=[
            row_spec(128), row_spec(128), row_spec(256),
            row_spec(128), row_spec(128),
            full_spec((32, 16)), full_spec((1, 16)),
            full_spec((16, 1)), full_spec((1, 1)),
        ],
        out_specs=pl.BlockSpec((br, 8), lambda i: (i, 0)),
        out_shape=jax.ShapeDtypeStruct((rp, 8), jnp.float32),
    )(g0.reshape(rp, 128), g1.reshape(rp, 128), r, a0_p, a1_p,
      bf(Wh1.T), bh1.reshape(1, 16), bf(Wh2.T), bh2.reshape(1, 1))

    return out8.reshape(n_pad)[:n]


# speedup vs baseline: 27.1853x; 1.0370x over previous
import functools

import jax
import jax.numpy as jnp
from jax import lax
from jax.experimental import pallas as pl
from jax.experimental.pallas import tpu as pltpu
from jax.experimental.pallas import tpu_sc as plsc

N_CORES = 2
N_SUBCORES = 16
LANES = 128
G = 5
FEAT = 16


def _seg_sum_body(edge_split, n_pad, n_rows,
                  src_hbm, dst_hbm, t0_hbm, t1_hbm, out0_hbm, out1_hbm,
                  srcv0, dstv0, rows0, srcv1, dstv1, rows1,
                  accsh, zbuf,
                  isem0, gsem0, ssem0, isem1, gsem1, ssem1):
    c = lax.axis_index("c")
    s = lax.axis_index("s")

    assert n_rows % G == 0
    total_slots = n_rows // G
    if edge_split:
        wid = s * N_CORES + c
        stride = N_CORES * N_SUBCORES
    else:
        wid = s
        stride = N_SUBCORES
    n_k = -(-total_slots // stride)
    n_k2 = -(-n_k // 2)

    srcv = (srcv0, srcv1)
    dstv = (dstv0, dstv1)
    rows = (rows0, rows1)
    isem = (isem0, isem1)
    gsem = (gsem0, gsem1)
    ssem = (ssem0, ssem1)

    tile_rows = n_pad // N_SUBCORES
    zrows = zbuf.shape[0]

    def zbody(i, carry):
        zbuf[i, :] = jnp.zeros((FEAT,), jnp.float32)
        return carry

    lax.fori_loop(0, zrows, zbody, 0)
    tile_base = s * tile_rows
    for kk in range(tile_rows // zrows):
        pltpu.sync_copy(zbuf, accsh.at[pl.ds(tile_base + kk * zrows, zrows)])
    plsc.subcore_barrier()

    def idx_issue(g, p):
        r0 = g * G
        pltpu.async_copy(src_hbm.at[pl.ds(r0, G)], srcv[p], isem[p])
        pltpu.async_copy(dst_hbm.at[pl.ds(r0, G)], dstv[p], isem[p])

    def idx_drain(p):
        pltpu.make_async_copy(src_hbm.at[pl.ds(0, G)], srcv[p], isem[p]).wait()
        pltpu.make_async_copy(dst_hbm.at[pl.ds(0, G)], dstv[p], isem[p]).wait()

    def gather_issue(table, p):
        for j in range(G):
            pltpu.async_copy(table.at[srcv[p].at[j]], rows[p].at[j], gsem[p])

    def gather_drain(table, p):
        for j in range(G):
            pltpu.make_async_copy(table.at[srcv[p].at[j]], rows[p].at[j],
                                  gsem[p]).wait()

    def scatter_issue(p):
        for j in range(G):
            pltpu.async_copy(rows[p].at[j], accsh.at[dstv[p].at[j]], ssem[p],
                             add=True)

    def scatter_drain(p):
        for j in range(G):
            pltpu.make_async_copy(rows[p].at[j], accsh.at[dstv[p].at[j]],
                                  ssem[p]).wait()

    def main_loop(table):
        @pl.when(wid < total_slots)
        def _():
            idx_issue(wid, 0)

        def kbody(k2, carry):
            for p in (0, 1):
                k = 2 * k2 + p
                g = wid + stride * k
                q = 1 - p
                g_prev = g - stride
                g_next = g + stride

                @pl.when(g < total_slots)
                def _():
                    idx_drain(p)
                    gather_issue(table, p)

                @pl.when((k >= 1) & (g_prev < total_slots))
                def _():
                    scatter_drain(q)

                @pl.when(g_next < total_slots)
                def _():
                    idx_issue(g_next, q)

                @pl.when(g < total_slots)
                def _():
                    gather_drain(table, p)
                    scatter_issue(p)

            return carry

        lax.fori_loop(0, n_k2, kbody, 0)

        k_last = 2 * n_k2 - 1

        @pl.when(wid + stride * k_last < total_slots)
        def _():
            scatter_drain(k_last % 2)

    @pl.when(c == 0)
    def _():
        main_loop(t0_hbm)

    @pl.when(c == 1)
    def _():
        main_loop(t1_hbm)

    plsc.subcore_barrier()

    @pl.when(c == 0)
    def _():
        pltpu.sync_copy(accsh.at[pl.ds(tile_base, tile_rows)],
                        out0_hbm.at[pl.ds(tile_base, tile_rows)])

    @pl.when(c == 1)
    def _():
        pltpu.sync_copy(accsh.at[pl.ds(tile_base, tile_rows)],
                        out1_hbm.at[pl.ds(tile_base, tile_rows)])


def _make_seg_sum(n_pad, n_rows, edge_split):
    mesh = plsc.VectorSubcoreMesh(core_axis_name="c", subcore_axis_name="s",
                                  num_cores=N_CORES, num_subcores=N_SUBCORES)
    tile_rows = n_pad // N_SUBCORES
    assert tile_rows % 16 == 0
    zrows = next(z for z in range(256, 7, -8) if tile_rows % z == 0)
    out = jax.ShapeDtypeStruct((n_pad, FEAT), jnp.float32)
    buf = [
        pltpu.VMEM((G, LANES), jnp.int32),
        pltpu.VMEM((G, LANES), jnp.int32),
        pltpu.VMEM((G, LANES, FEAT), jnp.float32),
    ]
    return pl.kernel(
        functools.partial(_seg_sum_body, edge_split, n_pad, n_rows),
        out_type=(out, out),
        mesh=mesh,
        scratch_types=buf + buf + [
            pltpu.VMEM_SHARED((n_pad, FEAT), jnp.float32),
            pltpu.VMEM((zrows, FEAT), jnp.float32),
        ] + [pltpu.SemaphoreType.DMA] * 6,
        compiler_params=pltpu.CompilerParams(use_tc_tiling_on_sc=False),
    )


def _bdot(a, b):
    return jnp.dot(a.astype(jnp.bfloat16), b, preferred_element_type=jnp.float32)


def _pack_body(xr_ref, out_ref):
    bR = out_ref.shape[0]
    out_ref[...] = jnp.zeros((bR, 128), jnp.float32)
    ones = jnp.ones((bR, 1), jnp.float32)
    for g in range(8):
        out_ref[:, 16 * g:16 * g + 8] = xr_ref[:, 8 * g:8 * g + 8]
        out_ref[:, 16 * g + 8:16 * g + 9] = ones


def _tc_a_body(xa_ref, a0_ref, a1_ref, w1l_ref, b1_ref, w1r_ref,
               w2lo_ref, w2hi_ref, b2_ref, w2r_ref,
               plo_ref, phi_ref, r_ref):
    a = a0_ref[...] + a1_ref[...]
    parts = []
    for g in range(8):
        sub = a[:, 16 * g:16 * g + 16]
        invd = 1.0 / jnp.maximum(sub[:, 8:9], 1.0)
        parts.append(sub * invd)
    as_p = jnp.concatenate(parts, axis=1)
    h = _bdot(as_p, w1l_ref[...]) + _bdot(xa_ref[...], w1r_ref[...])
    h = jnp.maximum(h + b1_ref[...], 0.0)
    plo_ref[...] = _bdot(h, w2lo_ref[...])
    phi_ref[...] = _bdot(h, w2hi_ref[...])
    r_ref[...] = _bdot(h, w2r_ref[...]) + b2_ref[...]


def _tc_b_body(g0_ref, g1_ref, r_ref, a0_ref, a1_ref, wh1_ref, bh1_ref,
               whd_ref, bh2_ref, out_ref):
    a = a0_ref[...] + a1_ref[...]
    g0 = g0_ref[...]
    g1 = g1_ref[...]
    parts = []
    for g in range(8):
        invd = 1.0 / jnp.maximum(a[:, 16 * g + 8:16 * g + 9], 1.0)
        parts.append(g0[:, 16 * g:16 * g + 16] * invd)
        parts.append(g1[:, 16 * g:16 * g + 16] * invd)
    agg = jnp.concatenate(parts, axis=1)
    z = jnp.maximum(agg + r_ref[...], 0.0)
    h1 = jnp.maximum(_bdot(z, wh1_ref[...]) + bh1_ref[...], 0.0)
    out_ref[...] = jax.nn.sigmoid(_bdot(h1, whd_ref[...]) + bh2_ref[...])


def kernel(x, edge_index, W1_l, b1_l, W1_r, W2_l, b2_l, W2_r, Wh1, bh1, Wh2, bh2):
    n = x.shape[0]
    e = edge_index.shape[1]
    assert e % LANES == 0
    n_rows = e // LANES
    n_pad = -(-n // 128) * 128

    ei_flat = edge_index.astype(jnp.int32).reshape(2 * e)
    src2d = ei_flat[:e].reshape(n_rows, LANES)
    dst2d = ei_flat[e:].reshape(n_rows, LANES)

    rp = n_pad // 8
    br = 3128
    assert rp % br == 0
    grid = (rp // br,)
    row_spec = lambda w: pl.BlockSpec((br, w), lambda i: (i, 0))
    full_spec = lambda shape: pl.BlockSpec(shape, lambda i: (0, 0))
    bf = lambda w: w.astype(jnp.bfloat16)

    xr = x.reshape(n // 8, 64)
    xr = jnp.concatenate(
        [xr, jnp.zeros((rp - n // 8, 64), jnp.float32)], axis=0)
    xa_p = pl.pallas_call(
        _pack_body,
        grid=grid,
        in_specs=[row_spec(64)],
        out_specs=row_spec(128),
        out_shape=jax.ShapeDtypeStruct((rp, 128), jnp.float32),
    )(xr)
    xa = xa_p.reshape(n_pad, FEAT)

    seg1 = _make_seg_sum(n_pad, n_rows, edge_split=True)
    a0, a1 = seg1(src2d, dst2d, xa, xa)
    a0_p = a0.reshape(rp, 128)
    a1_p = a1.reshape(rp, 128)

    eye8 = jnp.eye(8, dtype=jnp.float32)

    def bd16(w):
        return bf(jnp.kron(eye8, w))

    w16 = lambda wt: jnp.zeros((16, wt.shape[1]), jnp.float32).at[:8].set(wt)
    w1l_bd = bd16(w16(W1_l.T))
    w1r_bd = bd16(w16(W1_r.T))
    w2lo_bd = bf(jnp.kron(eye8, W2_l.T[:, :FEAT]))
    w2hi_bd = bf(jnp.kron(eye8, W2_l.T[:, FEAT:]))
    w2r_bd = bf(jnp.kron(eye8, W2_r.T))
    b1_t = jnp.tile(b1_l, 8).reshape(1, 512)
    b2_t = jnp.tile(b2_l, 8).reshape(1, 256)

    p_lo, p_hi, r = pl.pallas_call(
        _tc_a_body,
        grid=grid,
        in_specs=[
            row_spec(128), row_spec(128), row_spec(128),
            full_spec((128, 512)), full_spec((1, 512)), full_spec((128, 512)),
            full_spec((512, 128)), full_spec((512, 128)),
            full_spec((1, 256)), full_spec((512, 256)),
        ],
        out_specs=[row_spec(128), row_spec(128), row_spec(256)],
        out_shape=[
            jax.ShapeDtypeStruct((rp, 128), jnp.float32),
            jax.ShapeDtypeStruct((rp, 128), jnp.float32),
            jax.ShapeDtypeStruct((rp, 256), jnp.float32),
        ],
    )(xa_p, a0_p, a1_p, w1l_bd, b1_t, w1r_bd,
      w2lo_bd, w2hi_bd, b2_t, w2r_bd)

    seg2 = _make_seg_sum(n_pad, n_rows, edge_split=False)
    g0, g1 = seg2(src2d, dst2d,
                  p_lo.reshape(n_pad, FEAT), p_hi.reshape(n_pad, FEAT))

    wh1_bd = bf(jnp.kron(eye8, Wh1.T))
    whd_bd = bf(jnp.kron(eye8, Wh2.T))
    bh1_t = jnp.tile(bh1, 8).reshape(1, 128)

    out8 = pl.pallas_call(
        _tc_b_body,
        grid=grid,
        in_specs=[
            row_spec(128), row_spec(128), row_spec(256),
            row_spec(128), row_spec(128),
            full_spec((256, 128)), full_spec((1, 128)),
            full_spec((128, 8)), full_spec((1, 1)),
        ],
        out_specs=pl.BlockSpec((br, 8), lambda i: (i, 0)),
        out_shape=jax.ShapeDtypeStruct((rp, 8), jnp.float32),
    )(g0.reshape(rp, 128), g1.reshape(rp, 128), r, a0_p, a1_p,
      wh1_bd, bh1_t, whd_bd, bh2.reshape(1, 1))

    return out8.reshape(n_pad)[:n]


# speedup vs baseline: 32.2145x; 1.1850x over previous
import functools

import jax
import jax.numpy as jnp
from jax import lax
from jax.experimental import pallas as pl
from jax.experimental.pallas import tpu as pltpu
from jax.experimental.pallas import tpu_sc as plsc

N_CORES = 2
N_SUBCORES = 16
LANES = 128
G = 5
FEAT = 16


def _seg_sum_body(edge_split, n_pad, n_rows,
                  edges_hbm, t0_hbm, t1_hbm, out0_hbm, out1_hbm,
                  idxv0, rows0, idxv1, rows1,
                  accsh, zbuf,
                  isem0, gsem0, ssem0, isem1, gsem1, ssem1):
    c = lax.axis_index("c")
    s = lax.axis_index("s")

    assert n_rows % G == 0
    total_slots = n_rows // G
    if edge_split:
        wid = s * N_CORES + c
        stride = N_CORES * N_SUBCORES
    else:
        wid = s
        stride = N_SUBCORES
    n_k = -(-total_slots // stride)
    n_k2 = -(-n_k // 2)

    idxv = (idxv0, idxv1)
    rows = (rows0, rows1)
    isem = (isem0, isem1)
    gsem = (gsem0, gsem1)
    ssem = (ssem0, ssem1)

    tile_rows = n_pad // N_SUBCORES
    zrows = zbuf.shape[0]

    def zbody(i, carry):
        zbuf[i, :] = jnp.zeros((FEAT,), jnp.float32)
        return carry

    lax.fori_loop(0, zrows, zbody, 0)
    tile_base = s * tile_rows
    for kk in range(tile_rows // zrows):
        pltpu.sync_copy(zbuf, accsh.at[pl.ds(tile_base + kk * zrows, zrows)])
    plsc.subcore_barrier()

    def idx_issue(g, p):
        r0 = g * G
        pltpu.async_copy(edges_hbm.at[pl.ds(r0, G)], idxv[p], isem[p])

    def idx_drain(p):
        pltpu.make_async_copy(edges_hbm.at[pl.ds(0, G)], idxv[p], isem[p]).wait()

    def gather_issue(table, p):
        for j in range(G):
            pltpu.async_copy(table.at[idxv[p].at[j, 0]], rows[p].at[j], gsem[p])

    def gather_drain(table, p):
        for j in range(G):
            pltpu.make_async_copy(table.at[idxv[p].at[j, 0]], rows[p].at[j],
                                  gsem[p]).wait()

    def scatter_issue(p):
        for j in range(G):
            pltpu.async_copy(rows[p].at[j], accsh.at[idxv[p].at[j, 1]], ssem[p],
                             add=True)

    def scatter_drain(p):
        for j in range(G):
            pltpu.make_async_copy(rows[p].at[j], accsh.at[idxv[p].at[j, 1]],
                                  ssem[p]).wait()

    def main_loop(table):
        @pl.when(wid < total_slots)
        def _():
            idx_issue(wid, 0)

        def kbody(k2, carry):
            for p in (0, 1):
                k = 2 * k2 + p
                g = wid + stride * k
                q = 1 - p
                g_prev = g - stride
                g_next = g + stride

                @pl.when(g < total_slots)
                def _():
                    idx_drain(p)
                    gather_issue(table, p)

                @pl.when((k >= 1) & (g_prev < total_slots))
                def _():
                    scatter_drain(q)

                @pl.when(g_next < total_slots)
                def _():
                    idx_issue(g_next, q)

                @pl.when(g < total_slots)
                def _():
                    gather_drain(table, p)
                    scatter_issue(p)

            return carry

        lax.fori_loop(0, n_k2, kbody, 0)

        k_last = 2 * n_k2 - 1

        @pl.when(wid + stride * k_last < total_slots)
        def _():
            scatter_drain(k_last % 2)

    @pl.when(c == 0)
    def _():
        main_loop(t0_hbm)

    @pl.when(c == 1)
    def _():
        main_loop(t1_hbm)

    plsc.subcore_barrier()

    @pl.when(c == 0)
    def _():
        pltpu.sync_copy(accsh.at[pl.ds(tile_base, tile_rows)],
                        out0_hbm.at[pl.ds(tile_base, tile_rows)])

    @pl.when(c == 1)
    def _():
        pltpu.sync_copy(accsh.at[pl.ds(tile_base, tile_rows)],
                        out1_hbm.at[pl.ds(tile_base, tile_rows)])


def _make_seg_sum(n_pad, n_rows, edge_split):
    mesh = plsc.VectorSubcoreMesh(core_axis_name="c", subcore_axis_name="s",
                                  num_cores=N_CORES, num_subcores=N_SUBCORES)
    tile_rows = n_pad // N_SUBCORES
    assert tile_rows % 16 == 0
    zrows = next(z for z in range(256, 7, -8) if tile_rows % z == 0)
    out = jax.ShapeDtypeStruct((n_pad, FEAT), jnp.float32)
    buf = [
        pltpu.VMEM((G, 2, LANES), jnp.int32),
        pltpu.VMEM((G, LANES, FEAT), jnp.float32),
    ]
    return pl.kernel(
        functools.partial(_seg_sum_body, edge_split, n_pad, n_rows),
        out_type=(out, out),
        mesh=mesh,
        scratch_types=buf + buf + [
            pltpu.VMEM_SHARED((n_pad, FEAT), jnp.float32),
            pltpu.VMEM((zrows, FEAT), jnp.float32),
        ] + [pltpu.SemaphoreType.DMA] * 6,
        compiler_params=pltpu.CompilerParams(use_tc_tiling_on_sc=False),
    )


def _bdot(a, b):
    return jnp.dot(a.astype(jnp.bfloat16), b, preferred_element_type=jnp.float32)


def _pack_body(xr_ref, out_ref):
    bR = out_ref.shape[0]
    out_ref[...] = jnp.zeros((bR, 128), jnp.float32)
    ones = jnp.ones((bR, 1), jnp.float32)
    for g in range(8):
        out_ref[:, 16 * g:16 * g + 8] = xr_ref[:, 8 * g:8 * g + 8]
        out_ref[:, 16 * g + 8:16 * g + 9] = ones


def _tc_a_body(xa_ref, a0_ref, a1_ref, w1l_ref, b1_ref, w1r_ref,
               w2lo_ref, w2hi_ref, b2_ref, w2r_ref,
               plo_ref, phi_ref, r_ref):
    a = a0_ref[...] + a1_ref[...]
    parts = []
    for g in range(8):
        sub = a[:, 16 * g:16 * g + 16]
        invd = 1.0 / jnp.maximum(sub[:, 8:9], 1.0)
        parts.append(sub * invd)
    as_p = jnp.concatenate(parts, axis=1)
    h = _bdot(as_p, w1l_ref[...]) + _bdot(xa_ref[...], w1r_ref[...])
    h = jnp.maximum(h + b1_ref[...], 0.0)
    plo_ref[...] = _bdot(h, w2lo_ref[...])
    phi_ref[...] = _bdot(h, w2hi_ref[...])
    r_ref[...] = _bdot(h, w2r_ref[...]) + b2_ref[...]


def _tc_b_body(g0_ref, g1_ref, r_ref, a0_ref, a1_ref, wh1_ref, bh1_ref,
               whd_ref, bh2_ref, out_ref):
    a = a0_ref[...] + a1_ref[...]
    g0 = g0_ref[...]
    g1 = g1_ref[...]
    parts = []
    for g in range(8):
        invd = 1.0 / jnp.maximum(a[:, 16 * g + 8:16 * g + 9], 1.0)
        parts.append(g0[:, 16 * g:16 * g + 16] * invd)
        parts.append(g1[:, 16 * g:16 * g + 16] * invd)
    agg = jnp.concatenate(parts, axis=1)
    z = jnp.maximum(agg + r_ref[...], 0.0)
    h1 = jnp.maximum(_bdot(z, wh1_ref[...]) + bh1_ref[...], 0.0)
    out_ref[...] = jax.nn.sigmoid(_bdot(h1, whd_ref[...]) + bh2_ref[...])


def kernel(x, edge_index, W1_l, b1_l, W1_r, W2_l, b2_l, W2_r, Wh1, bh1, Wh2, bh2):
    n = x.shape[0]
    e = edge_index.shape[1]
    assert e % LANES == 0
    n_rows = e // LANES
    n_pad = -(-n // 128) * 128

    edges3 = (edge_index.astype(jnp.int32)
              .reshape(2, n_rows, LANES).transpose(1, 0, 2))

    rp = n_pad // 8
    br = 3128
    assert rp % br == 0
    grid = (rp // br,)
    row_spec = lambda w: pl.BlockSpec((br, w), lambda i: (i, 0))
    full_spec = lambda shape: pl.BlockSpec(shape, lambda i: (0, 0))
    bf = lambda w: w.astype(jnp.bfloat16)

    xr = x.reshape(n // 8, 64)
    xr = jnp.concatenate(
        [xr, jnp.zeros((rp - n // 8, 64), jnp.float32)], axis=0)
    xa_p = pl.pallas_call(
        _pack_body,
        grid=grid,
        in_specs=[row_spec(64)],
        out_specs=row_spec(128),
        out_shape=jax.ShapeDtypeStruct((rp, 128), jnp.float32),
    )(xr)
    xa = xa_p.reshape(n_pad, FEAT)

    seg1 = _make_seg_sum(n_pad, n_rows, edge_split=True)
    a0, a1 = seg1(edges3, xa, xa)
    a0_p = a0.reshape(rp, 128)
    a1_p = a1.reshape(rp, 128)

    eye8 = jnp.eye(8, dtype=jnp.float32)

    def bd16(w):
        return bf(jnp.kron(eye8, w))

    w16 = lambda wt: jnp.zeros((16, wt.shape[1]), jnp.float32).at[:8].set(wt)
    w1l_bd = bd16(w16(W1_l.T))
    w1r_bd = bd16(w16(W1_r.T))
    w2lo_bd = bf(jnp.kron(eye8, W2_l.T[:, :FEAT]))
    w2hi_bd = bf(jnp.kron(eye8, W2_l.T[:, FEAT:]))
    w2r_bd = bf(jnp.kron(eye8, W2_r.T))
    b1_t = jnp.tile(b1_l, 8).reshape(1, 512)
    b2_t = jnp.tile(b2_l, 8).reshape(1, 256)

    p_lo, p_hi, r = pl.pallas_call(
        _tc_a_body,
        grid=grid,
        in_specs=[
            row_spec(128), row_spec(128), row_spec(128),
            full_spec((128, 512)), full_spec((1, 512)), full_spec((128, 512)),
            full_spec((512, 128)), full_spec((512, 128)),
            full_spec((1, 256)), full_spec((512, 256)),
        ],
        out_specs=[row_spec(128), row_spec(128), row_spec(256)],
        out_shape=[
            jax.ShapeDtypeStruct((rp, 128), jnp.float32),
            jax.ShapeDtypeStruct((rp, 128), jnp.float32),
            jax.ShapeDtypeStruct((rp, 256), jnp.float32),
        ],
    )(xa_p, a0_p, a1_p, w1l_bd, b1_t, w1r_bd,
      w2lo_bd, w2hi_bd, b2_t, w2r_bd)

    seg2 = _make_seg_sum(n_pad, n_rows, edge_split=False)
    g0, g1 = seg2(edges3,
                  p_lo.reshape(n_pad, FEAT), p_hi.reshape(n_pad, FEAT))

    wh1_bd = bf(jnp.kron(eye8, Wh1.T))
    whd_bd = bf(jnp.kron(eye8, Wh2.T))
    bh1_t = jnp.tile(bh1, 8).reshape(1, 128)

    out8 = pl.pallas_call(
        _tc_b_body,
        grid=grid,
        in_specs=[
            row_spec(128), row_spec(128), row_spec(256),
            row_spec(128), row_spec(128),
            full_spec((256, 128)), full_spec((1, 128)),
            full_spec((128, 8)), full_spec((1, 1)),
        ],
        out_specs=pl.BlockSpec((br, 8), lambda i: (i, 0)),
        out_shape=jax.ShapeDtypeStruct((rp, 8), jnp.float32),
    )(g0.reshape(rp, 128), g1.reshape(rp, 128), r, a0_p, a1_p,
      wh1_bd, bh1_t, whd_bd, bh2.reshape(1, 1))

    return out8.reshape(n_pad)[:n]
